# baseline (speedup 1.0000x reference)
import functools

import jax
import jax.numpy as jnp
from jax import lax
from jax.experimental import pallas as pl
from jax.experimental.pallas import tpu as pltpu
from jax.experimental.pallas import tpu_sc as plsc

B = 16384
L = 16
NW = 32
RPW = B // NW
G = RPW // L
XW = 66
D = 16
NU = 1000000
NI = 100000

CHU = 2048
NFULL = NU // CHU
TAIL0 = NFULL * CHU
TAIL1 = TAIL0 + 512
BUFW = D * CHU

CB = 64
NCB = RPW // CB
UROWS = NU // L

_f32 = jnp.float32
_i32 = jnp.int32


def _detile_body(cfuT_hbm, nnuT_hbm, tseg_cfu, tseg_nnu, nnW_hbm, par_hbm,
                 cfu_f, nnd_f, buf, tbuf, par_v, nnW_v, semr, semw0, semw1):
    cid = lax.axis_index("c")
    sid = lax.axis_index("s")
    wid = cid * 16 + sid
    sems = (semw0, semw1)
    lanes = lax.iota(_i32, L)

    pltpu.sync_copy(par_hbm, par_v)
    pltpu.sync_copy(nnW_hbm, nnW_v)
    pa_nn = par_v[0, :]
    wnnu = jnp.zeros((L,), _f32)
    for k in range(16):
        wnnu = wnnu + pa_nn[k] * nnW_v[k, pl.ds(0, L)]

    def dot_cols(src, off, n, dst, dsl):
        def dot_g(g, carry):
            acc = jnp.zeros((L,), _f32)
            for d in range(D):
                acc = acc + wnnu[d] * src[pl.ds(off + d * n + g * L, L)]
            dst[pl.ds(dsl + g * L, L)] = acc
            return carry
        lax.fori_loop(0, n // L, dot_g, 0)

    def do_table(tT, tf, tseg, reduce_dot):
        wbytes = CHU if reduce_dot else BUFW

        def outer(i, carry):
            for b2 in range(2):
                c = (i * 2 + b2) * NW + wid
                u0 = c * CHU

                @pl.when(jnp.logical_and(c >= 2 * NW, c < NFULL))
                def _drain():
                    pltpu.make_async_copy(
                        tf.at[pl.ds(0, wbytes)],
                        buf.at[pl.ds(b2 * BUFW, wbytes)], sems[b2]).wait()

                @pl.when(c < NFULL)
                def _work():
                    rds = []
                    for d in range(D):
                        rds.append(pltpu.async_copy(
                            tT.at[d, pl.ds(u0, CHU)],
                            buf.at[pl.ds(b2 * BUFW + d * CHU, CHU)], semr))
                    for r in rds:
                        r.wait()
                    if reduce_dot:
                        dot_cols(buf, b2 * BUFW, CHU, buf,
                                 2 * BUFW + b2 * CHU)
                        pltpu.async_copy(
                            buf.at[pl.ds(2 * BUFW + b2 * CHU, CHU)],
                            tf.at[pl.ds(u0, CHU)], sems[b2])
                    else:
                        for d in range(D):
                            pltpu.async_copy(
                                buf.at[pl.ds(b2 * BUFW + d * CHU, CHU)],
                                tf.at[pl.ds(d * NU + u0, CHU)], sems[b2])
            return carry
        lax.fori_loop(0, (NFULL + 2 * NW - 1) // (2 * NW), outer, 0)
        for b2 in range(2):
            pltpu.make_async_copy(tf.at[pl.ds(0, wbytes)],
                                  buf.at[pl.ds(b2 * BUFW, wbytes)],
                                  sems[b2]).wait()

        @pl.when(wid == 0)
        def _tail0():
            rds = [pltpu.async_copy(tT.at[d, pl.ds(TAIL0, 512)],
                                    tbuf.at[pl.ds(d * 512, 512)], semr)
                   for d in range(D)]
            for r in rds:
                r.wait()
            if reduce_dot:
                dot_cols(tbuf, 0, 512, tbuf, D * 512)
                pltpu.async_copy(tbuf.at[pl.ds(D * 512, 512)],
                                 tf.at[pl.ds(TAIL0, 512)], semr).wait()
            else:
                wrs = [pltpu.async_copy(tbuf.at[pl.ds(d * 512, 512)],
                                        tf.at[pl.ds(d * NU + TAIL0, 512)],
                                        semr)
                       for d in range(D)]
                for w in wrs:
                    w.wait()

        @pl.when(wid == 1)
        def _tail1():
            rds = [pltpu.async_copy(tseg.at[pl.ds(d * 64, 64)],
                                    tbuf.at[pl.ds(d * 64, 64)], semr)
                   for d in range(D)]
            for r in rds:
                r.wait()
            if reduce_dot:
                dot_cols(tbuf, 0, 64, tbuf, D * 64)
                pltpu.async_copy(tbuf.at[pl.ds(D * 64, 64)],
                                 tf.at[pl.ds(TAIL1, 64)], semr).wait()
            else:
                wrs = [pltpu.async_copy(tbuf.at[pl.ds(d * 64, 64)],
                                        tf.at[pl.ds(d * NU + TAIL1, 64)],
                                        semr)
                       for d in range(D)]
                for w in wrs:
                    w.wait()

    do_table(cfuT_hbm, cfu_f, tseg_cfu, False)
    do_table(nnuT_hbm, nnd_f, tseg_nnu, True)


_detile = functools.partial(
    pl.kernel,
    out_type=(jax.ShapeDtypeStruct((NU * D,), _f32),
              jax.ShapeDtypeStruct((NU,), _f32)),
    mesh=plsc.VectorSubcoreMesh(core_axis_name="c", subcore_axis_name="s",
                                num_cores=2, num_subcores=16),
    compiler_params=pltpu.CompilerParams(needs_layout_passes=False,
                                         use_tc_tiling_on_sc=True),
    scratch_types=[
        pltpu.VMEM((2 * BUFW + 2 * CHU,), _f32),
        pltpu.VMEM((D * 512 + 512,), _f32),
        pltpu.VMEM((8, 16), _f32),
        pltpu.VMEM((16, 32), _f32),
        pltpu.SemaphoreType.DMA,
        pltpu.SemaphoreType.DMA,
        pltpu.SemaphoreType.DMA,
    ],
)(_detile_body)


def _main_body(xT_hbm, cfu_hbm, cfi_hbm, nnd_hbm, nni_hbm, nnW_hbm, icW_hbm,
               ucW_hbm, par_hbm, out_hbm,
               xT_v, ug_v, us_v, ig_v, cfu_b, nnd_b, cfi_b, nni_b, out_v,
               par_v, nnW_v, icW_v, ucW_v, sem):
    cid = lax.axis_index("c")
    sid = lax.axis_index("s")
    wid = cid * 16 + sid
    base = wid * RPW

    pltpu.sync_copy(par_hbm, par_v)
    pltpu.sync_copy(nnW_hbm, nnW_v)
    pltpu.sync_copy(icW_hbm, icW_v)
    pltpu.sync_copy(ucW_hbm, ucW_v)
    pltpu.sync_copy(xT_hbm.at[:, pl.ds(base, RPW)], xT_v)

    lanes = lax.iota(_i32, L)

    def build(g, carry):
        sl = pl.ds(g * L, L)
        u = xT_v[0, sl].astype(_i32)
        i = xT_v[1, sl].astype(_i32)
        uq = jnp.right_shift(u, 4)
        for d in range(D):
            ug_v[d, sl] = uq + d * UROWS
        us_v[sl] = jnp.bitwise_and(u, 15)
        ig_v[sl] = i
        return carry
    lax.fori_loop(0, G, build, 0)

    ci1 = pltpu.async_copy(cfi_hbm.at[ig_v], cfi_b, sem)
    ci2 = pltpu.async_copy(nni_hbm.at[ig_v], nni_b, sem)

    pa_nn = par_v[0, :]
    pa_ic = par_v[1, :]
    pa_uc = par_v[2, :]
    pa_ab = par_v[3, :]
    wnni = jnp.zeros((L,), _f32)
    wic0 = jnp.zeros((L,), _f32)
    wic1 = jnp.zeros((L,), _f32)
    wuc0 = jnp.zeros((L,), _f32)
    wuc1 = jnp.zeros((L,), _f32)
    for k in range(16):
        s_nn = pa_nn[k]
        wnni = wnni + s_nn * nnW_v[k, pl.ds(L, L)]
        s_ic = pa_ic[k]
        wic0 = wic0 + s_ic * icW_v[k, pl.ds(0, L)]
        wic1 = wic1 + s_ic * icW_v[k, pl.ds(L, L)]
        s_uc = pa_uc[k]
        wuc0 = wuc0 + s_uc * ucW_v[k, pl.ds(0, L)]
        wuc1 = wuc1 + s_uc * ucW_v[k, pl.ds(L, L)]
    wfeat = (wic0, wic1, wuc0, wuc1)

    alpha = pa_ab[0]
    bias = (pa_ab[1]
            + jnp.sum(pa_nn * par_v[4, :])
            + jnp.sum(pa_ic * par_v[5, :])
            + jnp.sum(pa_uc * par_v[6, :]))

    def feats(g, carry):
        sl = pl.ds(g * L, L)
        acc = bias + jnp.zeros((L,), _f32)
        for c in range(4):
            for dd in range(16):
                d = c * 16 + dd
                acc = acc + wfeat[c][dd] * xT_v[2 + d, sl]
        out_v[sl] = acc
        return carry
    lax.fori_loop(0, G, feats, 0)

    ci1.wait()
    ci2.wait()

    def chunk(ch, carry):
        cb = ch * CB
        isl = pl.ds(cb, CB)
        cps = [pltpu.async_copy(nnd_hbm.at[ug_v.at[0, isl]], nnd_b, sem)]
        for d in range(D):
            cps.append(pltpu.async_copy(
                cfu_hbm.at[ug_v.at[d, isl]], cfu_b.at[d], sem))
        for c in cps:
            c.wait()
        for g2 in range(CB // L):
            rl = g2 * L + lanes
            sl = pl.ds(cb + g2 * L, L)
            usub = us_v[sl]
            acc = out_v[sl] + plsc.load_gather(nnd_b, [rl, usub])
            cfacc = jnp.zeros((L,), _f32)
            for d in range(D):
                dcol = jnp.zeros((L,), _i32) + d
                cu = plsc.load_gather(cfu_b, [dcol, rl, usub])
                ci = plsc.load_gather(cfi_b, [cb + rl, dcol])
                ni = plsc.load_gather(nni_b, [cb + rl, dcol])
                cfacc = cfacc + cu * ci
                acc = acc + wnni[d] * ni
            out_v[sl] = acc + alpha * cfacc
        return carry
    lax.fori_loop(0, NCB, chunk, 0)

    pltpu.sync_copy(out_v, out_hbm.at[pl.ds(base, RPW)])


_main = functools.partial(
    pl.kernel,
    out_type=jax.ShapeDtypeStruct((B,), _f32),
    mesh=plsc.VectorSubcoreMesh(core_axis_name="c", subcore_axis_name="s",
                                num_cores=2, num_subcores=16),
    compiler_params=pltpu.CompilerParams(needs_layout_passes=False,
                                         use_tc_tiling_on_sc=False),
    scratch_types=[
        pltpu.VMEM((XW, RPW), _f32),
        pltpu.VMEM((D, RPW), _i32),
        pltpu.VMEM((RPW,), _i32),
        pltpu.VMEM((RPW,), _i32),
        pltpu.VMEM((D, CB, L), _f32),
        pltpu.VMEM((CB, L), _f32),
        pltpu.VMEM((RPW, D), _f32),
        pltpu.VMEM((RPW, D), _f32),
        pltpu.VMEM((RPW,), _f32),
        pltpu.VMEM((8, 16), _f32),
        pltpu.VMEM((16, 32), _f32),
        pltpu.VMEM((16, 32), _f32),
        pltpu.VMEM((16, 32), _f32),
        pltpu.SemaphoreType.DMA,
    ],
)(_main_body)


def kernel(x, cf_user_emb, cf_item_emb, nn_user_emb, nn_item_emb, nn_fc_W,
           nn_fc_b, ic_W, ic_b, uc_W, uc_b, fc_W, fc_b,
           item_context_features_in, user_context_features_in):
    row3 = jnp.concatenate([fc_W[0, 0:1], fc_b, jnp.zeros((14,), _f32)])
    params = jnp.stack([
        fc_W[0, 1:17], fc_W[0, 17:33], fc_W[0, 33:49], row3,
        nn_fc_b, ic_b, uc_b, jnp.zeros((16,), _f32),
    ])
    tseg_cfu = cf_user_emb[TAIL1:, :].T.reshape(-1)
    tseg_nnu = nn_user_emb[TAIL1:, :].T.reshape(-1)
    cfu_f, nnd_f = _detile(cf_user_emb.T, nn_user_emb.T, tseg_cfu, tseg_nnu,
                           nn_fc_W, params)
    out = _main(x.T, cfu_f.reshape(NU, D), cf_item_emb,
                nnd_f.reshape(NU // L, L), nn_item_emb,
                nn_fc_W, ic_W, uc_W, params)
    return out[:, None]


# speedup vs baseline: 1.0675x; 1.0675x over previous
import functools

import jax
import jax.numpy as jnp
from jax import lax
from jax.experimental import pallas as pl
from jax.experimental.pallas import tpu as pltpu
from jax.experimental.pallas import tpu_sc as plsc

B = 16384
L = 16
NW = 32
RPW = B // NW
G = RPW // L
XW = 66
D = 16
NU = 1000000
NI = 100000

CHU = 2048
NFULL = NU // CHU
TAIL0 = NFULL * CHU
TAIL1 = TAIL0 + 512
BUFW = D * CHU

CB = 64
NCB = RPW // CB
UROWS = NU // L

_f32 = jnp.float32
_i32 = jnp.int32


def _detile_body(cfuT_hbm, nnuT_hbm, tseg_cfu, tseg_nnu, nnW_hbm, par_hbm,
                 cfu_f, nnd_f, buf, tbuf, par_v, nnW_v, semr, semr2, semw0,
                 semw1):
    cid = lax.axis_index("c")
    sid = lax.axis_index("s")
    wid = cid * 16 + sid
    sems = (semw0, semw1)
    semrs = (semr, semr2)
    lanes = lax.iota(_i32, L)

    pltpu.sync_copy(par_hbm, par_v)
    pltpu.sync_copy(nnW_hbm, nnW_v)
    pa_nn = par_v[0, :]
    wnnu = jnp.zeros((L,), _f32)
    for k in range(16):
        wnnu = wnnu + pa_nn[k] * nnW_v[k, pl.ds(0, L)]

    def dot_cols(src, off, n, dst, dsl):
        def dot_g(g, carry):
            acc = jnp.zeros((L,), _f32)
            for d in range(D):
                acc = acc + wnnu[d] * src[pl.ds(off + d * n + g * L, L)]
            dst[pl.ds(dsl + g * L, L)] = acc
            return carry
        lax.fori_loop(0, n // L, dot_g, 0)

    def do_table(tT, tf, tseg, reduce_dot):
        wwords = CHU if reduce_dot else BUFW
        nslot = (NFULL + NW - 1) // NW

        def fire_reads(s):
            c = s * NW + wid
            b2 = s & 1

            @pl.when(c < NFULL)
            def _():
                for d in range(D):
                    pltpu.async_copy(
                        tT.at[d, pl.ds(c * CHU, CHU)],
                        buf.at[pl.ds(b2 * BUFW + d * CHU, CHU)], semrs[b2])

        def drain_writes(s):
            cprev = (s - 2) * NW + wid
            b2 = s & 1

            @pl.when(cprev < NFULL)
            def _():
                pltpu.make_async_copy(
                    tf.at[pl.ds(0, wwords)],
                    buf.at[pl.ds(b2 * BUFW, wwords)], sems[b2]).wait()

        def process(s):
            c = s * NW + wid
            b2 = s & 1

            @pl.when(c < NFULL)
            def _():
                pltpu.make_async_copy(
                    tf.at[pl.ds(0, BUFW)],
                    buf.at[pl.ds(b2 * BUFW, BUFW)], semrs[b2]).wait()
                if reduce_dot:
                    dot_cols(buf, b2 * BUFW, CHU, buf, 2 * BUFW + b2 * CHU)
                    pltpu.async_copy(
                        buf.at[pl.ds(2 * BUFW + b2 * CHU, CHU)],
                        tf.at[pl.ds(c * CHU, CHU)], sems[b2])
                else:
                    for d in range(D):
                        pltpu.async_copy(
                            buf.at[pl.ds(b2 * BUFW + d * CHU, CHU)],
                            tf.at[pl.ds(d * NU + c * CHU, CHU)], sems[b2])

        for s in range(nslot):
            if s >= 2:
                drain_writes(s)
            fire_reads(s)
            if s >= 1:
                process(s - 1)
        process(nslot - 1)
        pltpu.make_async_copy(tf.at[pl.ds(0, wwords)],
                              buf.at[pl.ds(0, wwords)], sems[0]).wait()
        clast = (nslot - 1) * NW + wid

        @pl.when(clast < NFULL)
        def _dlast():
            pltpu.make_async_copy(tf.at[pl.ds(0, wwords)],
                                  buf.at[pl.ds(BUFW, wwords)], sems[1]).wait()

        @pl.when(wid == 0)
        def _tail0():
            rds = [pltpu.async_copy(tT.at[d, pl.ds(TAIL0, 512)],
                                    tbuf.at[pl.ds(d * 512, 512)], semr)
                   for d in range(D)]
            for r in rds:
                r.wait()
            if reduce_dot:
                dot_cols(tbuf, 0, 512, tbuf, D * 512)
                pltpu.async_copy(tbuf.at[pl.ds(D * 512, 512)],
                                 tf.at[pl.ds(TAIL0, 512)], semr).wait()
            else:
                wrs = [pltpu.async_copy(tbuf.at[pl.ds(d * 512, 512)],
                                        tf.at[pl.ds(d * NU + TAIL0, 512)],
                                        semr)
                       for d in range(D)]
                for w in wrs:
                    w.wait()

        @pl.when(wid == 1)
        def _tail1():
            rds = [pltpu.async_copy(tseg.at[pl.ds(d * 64, 64)],
                                    tbuf.at[pl.ds(d * 64, 64)], semr)
                   for d in range(D)]
            for r in rds:
                r.wait()
            if reduce_dot:
                dot_cols(tbuf, 0, 64, tbuf, D * 64)
                pltpu.async_copy(tbuf.at[pl.ds(D * 64, 64)],
                                 tf.at[pl.ds(TAIL1, 64)], semr).wait()
            else:
                wrs = [pltpu.async_copy(tbuf.at[pl.ds(d * 64, 64)],
                                        tf.at[pl.ds(d * NU + TAIL1, 64)],
                                        semr)
                       for d in range(D)]
                for w in wrs:
                    w.wait()

    do_table(cfuT_hbm, cfu_f, tseg_cfu, False)
    do_table(nnuT_hbm, nnd_f, tseg_nnu, True)


_detile = functools.partial(
    pl.kernel,
    out_type=(jax.ShapeDtypeStruct((NU * D,), _f32),
              jax.ShapeDtypeStruct((NU,), _f32)),
    mesh=plsc.VectorSubcoreMesh(core_axis_name="c", subcore_axis_name="s",
                                num_cores=2, num_subcores=16),
    compiler_params=pltpu.CompilerParams(needs_layout_passes=False,
                                         use_tc_tiling_on_sc=True),
    scratch_types=[
        pltpu.VMEM((2 * BUFW + 2 * CHU,), _f32),
        pltpu.VMEM((D * 512 + 512,), _f32),
        pltpu.VMEM((8, 16), _f32),
        pltpu.VMEM((16, 32), _f32),
        pltpu.SemaphoreType.DMA,
        pltpu.SemaphoreType.DMA,
        pltpu.SemaphoreType.DMA,
        pltpu.SemaphoreType.DMA,
    ],
)(_detile_body)


def _main_body(xT_hbm, cfu_hbm, cfi_hbm, nnd_hbm, nni_hbm, nnW_hbm, icW_hbm,
               ucW_hbm, par_hbm, out_hbm,
               xT_v, ug_v, us_v, ig_v, cfu_b, nnd_b, cfi_b, nni_b, out_v,
               par_v, nnW_v, icW_v, ucW_v, sem):
    cid = lax.axis_index("c")
    sid = lax.axis_index("s")
    wid = cid * 16 + sid
    base = wid * RPW

    pltpu.sync_copy(par_hbm, par_v)
    pltpu.sync_copy(nnW_hbm, nnW_v)
    pltpu.sync_copy(icW_hbm, icW_v)
    pltpu.sync_copy(ucW_hbm, ucW_v)
    pltpu.sync_copy(xT_hbm.at[:, pl.ds(base, RPW)], xT_v)

    lanes = lax.iota(_i32, L)

    def build(g, carry):
        sl = pl.ds(g * L, L)
        u = xT_v[0, sl].astype(_i32)
        i = xT_v[1, sl].astype(_i32)
        uq = jnp.right_shift(u, 4)
        for d in range(D):
            ug_v[d, sl] = uq + d * UROWS
        us_v[sl] = jnp.bitwise_and(u, 15)
        ig_v[sl] = i
        return carry
    lax.fori_loop(0, G, build, 0)

    ci1 = pltpu.async_copy(cfi_hbm.at[ig_v], cfi_b, sem)
    ci2 = pltpu.async_copy(nni_hbm.at[ig_v], nni_b, sem)

    pa_nn = par_v[0, :]
    pa_ic = par_v[1, :]
    pa_uc = par_v[2, :]
    pa_ab = par_v[3, :]
    wnni = jnp.zeros((L,), _f32)
    wic0 = jnp.zeros((L,), _f32)
    wic1 = jnp.zeros((L,), _f32)
    wuc0 = jnp.zeros((L,), _f32)
    wuc1 = jnp.zeros((L,), _f32)
    for k in range(16):
        s_nn = pa_nn[k]
        wnni = wnni + s_nn * nnW_v[k, pl.ds(L, L)]
        s_ic = pa_ic[k]
        wic0 = wic0 + s_ic * icW_v[k, pl.ds(0, L)]
        wic1 = wic1 + s_ic * icW_v[k, pl.ds(L, L)]
        s_uc = pa_uc[k]
        wuc0 = wuc0 + s_uc * ucW_v[k, pl.ds(0, L)]
        wuc1 = wuc1 + s_uc * ucW_v[k, pl.ds(L, L)]
    wfeat = (wic0, wic1, wuc0, wuc1)

    alpha = pa_ab[0]
    bias = (pa_ab[1]
            + jnp.sum(pa_nn * par_v[4, :])
            + jnp.sum(pa_ic * par_v[5, :])
            + jnp.sum(pa_uc * par_v[6, :]))

    def feats(g, carry):
        sl = pl.ds(g * L, L)
        acc = bias + jnp.zeros((L,), _f32)
        for c in range(4):
            for dd in range(16):
                d = c * 16 + dd
                acc = acc + wfeat[c][dd] * xT_v[2 + d, sl]
        out_v[sl] = acc
        return carry
    lax.fori_loop(0, G, feats, 0)

    ci1.wait()
    ci2.wait()

    def chunk(ch, carry):
        cb = ch * CB
        isl = pl.ds(cb, CB)
        cps = [pltpu.async_copy(nnd_hbm.at[ug_v.at[0, isl]], nnd_b, sem)]
        for d in range(D):
            cps.append(pltpu.async_copy(
                cfu_hbm.at[ug_v.at[d, isl]], cfu_b.at[d], sem))
        for c in cps:
            c.wait()
        for g2 in range(CB // L):
            rl = g2 * L + lanes
            sl = pl.ds(cb + g2 * L, L)
            usub = us_v[sl]
            acc = out_v[sl] + plsc.load_gather(nnd_b, [rl, usub])
            cfacc = jnp.zeros((L,), _f32)
            for d in range(D):
                dcol = jnp.zeros((L,), _i32) + d
                cu = plsc.load_gather(cfu_b, [dcol, rl, usub])
                ci = plsc.load_gather(cfi_b, [cb + rl, dcol])
                ni = plsc.load_gather(nni_b, [cb + rl, dcol])
                cfacc = cfacc + cu * ci
                acc = acc + wnni[d] * ni
            out_v[sl] = acc + alpha * cfacc
        return carry
    lax.fori_loop(0, NCB, chunk, 0)

    pltpu.sync_copy(out_v, out_hbm.at[pl.ds(base, RPW)])


_main = functools.partial(
    pl.kernel,
    out_type=jax.ShapeDtypeStruct((B,), _f32),
    mesh=plsc.VectorSubcoreMesh(core_axis_name="c", subcore_axis_name="s",
                                num_cores=2, num_subcores=16),
    compiler_params=pltpu.CompilerParams(needs_layout_passes=False,
                                         use_tc_tiling_on_sc=False),
    scratch_types=[
        pltpu.VMEM((XW, RPW), _f32),
        pltpu.VMEM((D, RPW), _i32),
        pltpu.VMEM((RPW,), _i32),
        pltpu.VMEM((RPW,), _i32),
        pltpu.VMEM((D, CB, L), _f32),
        pltpu.VMEM((CB, L), _f32),
        pltpu.VMEM((RPW, D), _f32),
        pltpu.VMEM((RPW, D), _f32),
        pltpu.VMEM((RPW,), _f32),
        pltpu.VMEM((8, 16), _f32),
        pltpu.VMEM((16, 32), _f32),
        pltpu.VMEM((16, 32), _f32),
        pltpu.VMEM((16, 32), _f32),
        pltpu.SemaphoreType.DMA,
    ],
)(_main_body)


def kernel(x, cf_user_emb, cf_item_emb, nn_user_emb, nn_item_emb, nn_fc_W,
           nn_fc_b, ic_W, ic_b, uc_W, uc_b, fc_W, fc_b,
           item_context_features_in, user_context_features_in):
    row3 = jnp.concatenate([fc_W[0, 0:1], fc_b, jnp.zeros((14,), _f32)])
    params = jnp.stack([
        fc_W[0, 1:17], fc_W[0, 17:33], fc_W[0, 33:49], row3,
        nn_fc_b, ic_b, uc_b, jnp.zeros((16,), _f32),
    ])
    tseg_cfu = cf_user_emb[TAIL1:, :].T.reshape(-1)
    tseg_nnu = nn_user_emb[TAIL1:, :].T.reshape(-1)
    cfu_f, nnd_f = _detile(cf_user_emb.T, nn_user_emb.T, tseg_cfu, tseg_nnu,
                           nn_fc_W, params)
    out = _main(x.T, cfu_f.reshape(NU, D), cf_item_emb,
                nnd_f.reshape(NU // L, L), nn_item_emb,
                nn_fc_W, ic_W, uc_W, params)
    return out[:, None]


# speedup vs baseline: 1.1475x; 1.0749x over previous
import functools

import jax
import jax.numpy as jnp
from jax import lax
from jax.experimental import pallas as pl
from jax.experimental.pallas import tpu as pltpu
from jax.experimental.pallas import tpu_sc as plsc

B = 16384
L = 16
NW = 32
RPW = B // NW
G = RPW // L
XW = 66
D = 16
NU = 1000000
NI = 100000

CHU = 2048
NFULL = NU // CHU
TAIL0 = NFULL * CHU
TAIL1 = TAIL0 + 512
BUFW = D * CHU

CB = 64
NCB = RPW // CB
UROWS = NU // L

_f32 = jnp.float32
_i32 = jnp.int32


def _detile_body(cfuT_hbm, nnuT_hbm, nniT_hbm, tseg_cfu, tseg_nnu, tseg_nni,
                 nnW_hbm, par_hbm, cfu_f, nnd_f, nnid_f, buf, tbuf, par_v,
                 nnW_v, semr, semr2, semw0, semw1):
    cid = lax.axis_index("c")
    sid = lax.axis_index("s")
    wid = cid * 16 + sid
    sems = (semw0, semw1)
    semrs = (semr, semr2)

    pltpu.sync_copy(par_hbm, par_v)
    pltpu.sync_copy(nnW_hbm, nnW_v)
    pa_nn = par_v[0, :]
    wnnu = jnp.zeros((L,), _f32)
    wnni = jnp.zeros((L,), _f32)
    for k in range(16):
        wnnu = wnnu + pa_nn[k] * nnW_v[k, pl.ds(0, L)]
        wnni = wnni + pa_nn[k] * nnW_v[k, pl.ds(L, L)]

    def dot_cols(w, src, off, n, dst, dsl):
        def dot_g(g, carry):
            acc = jnp.zeros((L,), _f32)
            for d in range(D):
                acc = acc + w[d] * src[pl.ds(off + d * n + g * L, L)]
            dst[pl.ds(dsl + g * L, L)] = acc
            return carry
        lax.fori_loop(0, n // L, dot_g, 0)

    def do_table(tT, tf, tseg, rw, nfull, nrows, t0_off, t0_w, t1_off, t1_w):
        wwords = CHU if rw is not None else BUFW
        nslot = (nfull + NW - 1) // NW

        def fire_reads(s):
            c = s * NW + wid
            b2 = s & 1

            @pl.when(c < nfull)
            def _():
                for d in range(D):
                    pltpu.async_copy(
                        tT.at[d, pl.ds(c * CHU, CHU)],
                        buf.at[pl.ds(b2 * BUFW + d * CHU, CHU)], semrs[b2])

        def drain_writes(s):
            cprev = (s - 2) * NW + wid
            b2 = s & 1

            @pl.when(cprev < nfull)
            def _():
                pltpu.make_async_copy(
                    tf.at[pl.ds(0, wwords)],
                    buf.at[pl.ds(b2 * BUFW, wwords)], sems[b2]).wait()

        def process(s):
            c = s * NW + wid
            b2 = s & 1

            @pl.when(c < nfull)
            def _():
                pltpu.make_async_copy(
                    tf.at[pl.ds(0, BUFW)],
                    buf.at[pl.ds(b2 * BUFW, BUFW)], semrs[b2]).wait()
                if rw is not None:
                    dot_cols(rw, buf, b2 * BUFW, CHU, buf,
                             2 * BUFW + b2 * CHU)
                    pltpu.async_copy(
                        buf.at[pl.ds(2 * BUFW + b2 * CHU, CHU)],
                        tf.at[pl.ds(c * CHU, CHU)], sems[b2])
                else:
                    for d in range(D):
                        pltpu.async_copy(
                            buf.at[pl.ds(b2 * BUFW + d * CHU, CHU)],
                            tf.at[pl.ds(d * nrows + c * CHU, CHU)], sems[b2])

        for s in range(nslot):
            if s >= 2:
                drain_writes(s)
            fire_reads(s)
            if s >= 1:
                process(s - 1)
        process(nslot - 1)
        for sl in (nslot - 2, nslot - 1):
            cl = sl * NW + wid
            b2 = sl & 1

            @pl.when(cl < nfull)
            def _dl():
                pltpu.make_async_copy(
                    tf.at[pl.ds(0, wwords)],
                    buf.at[pl.ds(b2 * BUFW, wwords)], sems[b2]).wait()

        @pl.when(wid == 0)
        def _tail0():
            rds = [pltpu.async_copy(tT.at[d, pl.ds(t0_off, t0_w)],
                                    tbuf.at[pl.ds(d * t0_w, t0_w)], semr)
                   for d in range(D)]
            for r in rds:
                r.wait()
            if rw is not None:
                dot_cols(rw, tbuf, 0, t0_w, tbuf, D * t0_w)
                pltpu.async_copy(tbuf.at[pl.ds(D * t0_w, t0_w)],
                                 tf.at[pl.ds(t0_off, t0_w)], semr).wait()
            else:
                wrs = [pltpu.async_copy(tbuf.at[pl.ds(d * t0_w, t0_w)],
                                        tf.at[pl.ds(d * nrows + t0_off, t0_w)],
                                        semr)
                       for d in range(D)]
                for w in wrs:
                    w.wait()

        @pl.when(wid == 1)
        def _tail1():
            rds = [pltpu.async_copy(tseg.at[pl.ds(d * t1_w, t1_w)],
                                    tbuf.at[pl.ds(d * t1_w, t1_w)], semr)
                   for d in range(D)]
            for r in rds:
                r.wait()
            if rw is not None:
                dot_cols(rw, tbuf, 0, t1_w, tbuf, D * t1_w)
                pltpu.async_copy(tbuf.at[pl.ds(D * t1_w, t1_w)],
                                 tf.at[pl.ds(t1_off, t1_w)], semr).wait()
            else:
                wrs = [pltpu.async_copy(tbuf.at[pl.ds(d * t1_w, t1_w)],
                                        tf.at[pl.ds(d * nrows + t1_off, t1_w)],
                                        semr)
                       for d in range(D)]
                for w in wrs:
                    w.wait()

    do_table(cfuT_hbm, cfu_f, tseg_cfu, None, NFULL, NU, TAIL0, 512,
             TAIL1, 64)
    do_table(nnuT_hbm, nnd_f, tseg_nnu, wnnu, NFULL, NU, TAIL0, 512,
             TAIL1, 64)
    do_table(nniT_hbm, nnid_f, tseg_nni, wnni, NI // CHU, NI, (NI // CHU) * CHU,
             1664, 99968, 32)


_detile = functools.partial(
    pl.kernel,
    out_type=(jax.ShapeDtypeStruct((NU * D,), _f32),
              jax.ShapeDtypeStruct((NU,), _f32),
              jax.ShapeDtypeStruct((NI,), _f32)),
    mesh=plsc.VectorSubcoreMesh(core_axis_name="c", subcore_axis_name="s",
                                num_cores=2, num_subcores=16),
    compiler_params=pltpu.CompilerParams(needs_layout_passes=False,
                                         use_tc_tiling_on_sc=True),
    scratch_types=[
        pltpu.VMEM((2 * BUFW + 2 * CHU,), _f32),
        pltpu.VMEM((D * 1664 + 1664,), _f32),
        pltpu.VMEM((8, 16), _f32),
        pltpu.VMEM((16, 32), _f32),
        pltpu.SemaphoreType.DMA,
        pltpu.SemaphoreType.DMA,
        pltpu.SemaphoreType.DMA,
        pltpu.SemaphoreType.DMA,
    ],
)(_detile_body)


def _main_body(xT_hbm, cfu_hbm, cfi_hbm, nnd_hbm, nnid_hbm, icW_hbm,
               ucW_hbm, par_hbm, out_hbm,
               xT_v, ug_v, us_v, ig_v, iq_v, isub_v, cfu_b, nnd_b, nnid_b,
               cfi_b, out_v, par_v, icW_v, ucW_v, sem):
    cid = lax.axis_index("c")
    sid = lax.axis_index("s")
    wid = cid * 16 + sid
    base = wid * RPW

    pltpu.sync_copy(par_hbm, par_v)
    pltpu.sync_copy(icW_hbm, icW_v)
    pltpu.sync_copy(ucW_hbm, ucW_v)
    pltpu.sync_copy(xT_hbm.at[:, pl.ds(base, RPW)], xT_v)

    lanes = lax.iota(_i32, L)

    def build(g, carry):
        sl = pl.ds(g * L, L)
        u = xT_v[0, sl].astype(_i32)
        i = xT_v[1, sl].astype(_i32)
        uq = jnp.right_shift(u, 4)
        for d in range(D):
            ug_v[d, sl] = uq + d * UROWS
        us_v[sl] = jnp.bitwise_and(u, 15)
        ig_v[sl] = i
        iq_v[sl] = jnp.right_shift(i, 4)
        isub_v[sl] = jnp.bitwise_and(i, 15)
        return carry
    lax.fori_loop(0, G, build, 0)

    ci1 = pltpu.async_copy(cfi_hbm.at[ig_v], cfi_b, sem)
    ci2 = pltpu.async_copy(nnid_hbm.at[iq_v], nnid_b, sem)

    pa_nn = par_v[0, :]
    pa_ic = par_v[1, :]
    pa_uc = par_v[2, :]
    pa_ab = par_v[3, :]
    wic0 = jnp.zeros((L,), _f32)
    wic1 = jnp.zeros((L,), _f32)
    wuc0 = jnp.zeros((L,), _f32)
    wuc1 = jnp.zeros((L,), _f32)
    for k in range(16):
        s_ic = pa_ic[k]
        wic0 = wic0 + s_ic * icW_v[k, pl.ds(0, L)]
        wic1 = wic1 + s_ic * icW_v[k, pl.ds(L, L)]
        s_uc = pa_uc[k]
        wuc0 = wuc0 + s_uc * ucW_v[k, pl.ds(0, L)]
        wuc1 = wuc1 + s_uc * ucW_v[k, pl.ds(L, L)]
    wfeat = (wic0, wic1, wuc0, wuc1)

    alpha = pa_ab[0]
    bias = (pa_ab[1]
            + jnp.sum(pa_nn * par_v[4, :])
            + jnp.sum(pa_ic * par_v[5, :])
            + jnp.sum(pa_uc * par_v[6, :]))

    def feats(g, carry):
        sl = pl.ds(g * L, L)
        acc = bias + jnp.zeros((L,), _f32)
        for c in range(4):
            for dd in range(16):
                d = c * 16 + dd
                acc = acc + wfeat[c][dd] * xT_v[2 + d, sl]
        out_v[sl] = acc
        return carry
    lax.fori_loop(0, G, feats, 0)

    ci1.wait()
    ci2.wait()

    def chunk(ch, carry):
        cb = ch * CB
        isl = pl.ds(cb, CB)
        cps = [pltpu.async_copy(nnd_hbm.at[ug_v.at[0, isl]], nnd_b, sem)]
        for d in range(D):
            cps.append(pltpu.async_copy(
                cfu_hbm.at[ug_v.at[d, isl]], cfu_b.at[d], sem))
        for c in cps:
            c.wait()
        for g2 in range(CB // L):
            rl = g2 * L + lanes
            sl = pl.ds(cb + g2 * L, L)
            usub = us_v[sl]
            acc = (out_v[sl] + plsc.load_gather(nnd_b, [rl, usub])
                   + plsc.load_gather(nnid_b, [cb + rl, isub_v[sl]]))
            cfacc = jnp.zeros((L,), _f32)
            for d in range(D):
                dcol = jnp.zeros((L,), _i32) + d
                cu = plsc.load_gather(cfu_b, [dcol, rl, usub])
                ci = plsc.load_gather(cfi_b, [cb + rl, dcol])
                cfacc = cfacc + cu * ci
            out_v[sl] = acc + alpha * cfacc
        return carry
    lax.fori_loop(0, NCB, chunk, 0)

    pltpu.sync_copy(out_v, out_hbm.at[pl.ds(base, RPW)])


_main = functools.partial(
    pl.kernel,
    out_type=jax.ShapeDtypeStruct((B,), _f32),
    mesh=plsc.VectorSubcoreMesh(core_axis_name="c", subcore_axis_name="s",
                                num_cores=2, num_subcores=16),
    compiler_params=pltpu.CompilerParams(needs_layout_passes=False,
                                         use_tc_tiling_on_sc=False),
    scratch_types=[
        pltpu.VMEM((XW, RPW), _f32),
        pltpu.VMEM((D, RPW), _i32),
        pltpu.VMEM((RPW,), _i32),
        pltpu.VMEM((RPW,), _i32),
        pltpu.VMEM((RPW,), _i32),
        pltpu.VMEM((RPW,), _i32),
        pltpu.VMEM((D, CB, L), _f32),
        pltpu.VMEM((CB, L), _f32),
        pltpu.VMEM((RPW, L), _f32),
        pltpu.VMEM((RPW, D), _f32),
        pltpu.VMEM((RPW,), _f32),
        pltpu.VMEM((8, 16), _f32),
        pltpu.VMEM((16, 32), _f32),
        pltpu.VMEM((16, 32), _f32),
        pltpu.SemaphoreType.DMA,
    ],
)(_main_body)


def kernel(x, cf_user_emb, cf_item_emb, nn_user_emb, nn_item_emb, nn_fc_W,
           nn_fc_b, ic_W, ic_b, uc_W, uc_b, fc_W, fc_b,
           item_context_features_in, user_context_features_in):
    row3 = jnp.concatenate([fc_W[0, 0:1], fc_b, jnp.zeros((14,), _f32)])
    params = jnp.stack([
        fc_W[0, 1:17], fc_W[0, 17:33], fc_W[0, 33:49], row3,
        nn_fc_b, ic_b, uc_b, jnp.zeros((16,), _f32),
    ])
    tseg_cfu = cf_user_emb[TAIL1:, :].T.reshape(-1)
    tseg_nnu = nn_user_emb[TAIL1:, :].T.reshape(-1)
    tseg_nni = nn_item_emb[99968:, :].T.reshape(-1)
    cfu_f, nnd_f, nnid_f = _detile(cf_user_emb.T, nn_user_emb.T,
                                   nn_item_emb.T, tseg_cfu, tseg_nnu,
                                   tseg_nni, nn_fc_W, params)
    out = _main(x.T, cfu_f.reshape(NU, D), cf_item_emb,
                nnd_f.reshape(NU // L, L), nnid_f.reshape(NI // L, L),
                ic_W, uc_W, params)
    return out[:, None]


# speedup vs baseline: 1.1545x; 1.0061x over previous
import functools

import jax
import jax.numpy as jnp
from jax import lax
from jax.experimental import pallas as pl
from jax.experimental.pallas import tpu as pltpu
from jax.experimental.pallas import tpu_sc as plsc

B = 16384
L = 16
NW = 32
RPW = B // NW
G = RPW // L
XW = 66
D = 16
NU = 1000000
NI = 100000

CHU = 2048
NFULL = NU // CHU
TAIL0 = NFULL * CHU
TAIL1 = TAIL0 + 512
BUFW = D * CHU

CHI = 1024
BUFI = D * CHI
NFULL_I = NI // CHI
T0I_OFF = NFULL_I * CHI
T1I_OFF = 99968

CB = 64
NCB = RPW // CB
UROWS = NU // L

_f32 = jnp.float32
_i32 = jnp.int32


def _detile_body(cfuT_hbm, nnuT_hbm, nniT_hbm, cfiT_hbm, tseg_cfu, tseg_nnu,
                 tseg_nni, tseg_cfi, nnW_hbm, par_hbm, cfu_f, nnd_f, nnid_f,
                 cfi_f, buf, tbuf, par_v, nnW_v, semr, semr2, semw0, semw1):
    cid = lax.axis_index("c")
    sid = lax.axis_index("s")
    wid = cid * 16 + sid
    sems = (semw0, semw1)
    semrs = (semr, semr2)

    pltpu.sync_copy(par_hbm, par_v)
    pltpu.sync_copy(nnW_hbm, nnW_v)
    pa_nn = par_v[0, :]
    wnnu = jnp.zeros((L,), _f32)
    wnni = jnp.zeros((L,), _f32)
    for k in range(16):
        wnnu = wnnu + pa_nn[k] * nnW_v[k, pl.ds(0, L)]
        wnni = wnni + pa_nn[k] * nnW_v[k, pl.ds(L, L)]

    def dot_cols(w, src, off, n, dst, dsl):
        def dot_g(g, carry):
            acc = jnp.zeros((L,), _f32)
            for d in range(D):
                acc = acc + w[d] * src[pl.ds(off + d * n + g * L, L)]
            dst[pl.ds(dsl + g * L, L)] = acc
            return carry
        lax.fori_loop(0, n // L, dot_g, 0)

    def do_table(tT, tf, tseg, rw, nfull, nrows, t0_off, t0_w, t1_off, t1_w):
        wwords = CHU if rw is not None else BUFW
        nslot = (nfull + NW - 1) // NW

        def fire_reads(s):
            c = s * NW + wid
            b2 = s & 1

            @pl.when(c < nfull)
            def _():
                for d in range(D):
                    pltpu.async_copy(
                        tT.at[d, pl.ds(c * CHU, CHU)],
                        buf.at[pl.ds(b2 * BUFW + d * CHU, CHU)], semrs[b2])

        def drain_writes(s):
            cprev = (s - 2) * NW + wid
            b2 = s & 1

            @pl.when(cprev < nfull)
            def _():
                pltpu.make_async_copy(
                    tf.at[pl.ds(0, wwords)],
                    buf.at[pl.ds(b2 * BUFW, wwords)], sems[b2]).wait()

        def process(s):
            c = s * NW + wid
            b2 = s & 1

            @pl.when(c < nfull)
            def _():
                pltpu.make_async_copy(
                    tf.at[pl.ds(0, BUFW)],
                    buf.at[pl.ds(b2 * BUFW, BUFW)], semrs[b2]).wait()
                if rw is not None:
                    dot_cols(rw, buf, b2 * BUFW, CHU, buf,
                             2 * BUFW + b2 * CHU)
                    pltpu.async_copy(
                        buf.at[pl.ds(2 * BUFW + b2 * CHU, CHU)],
                        tf.at[pl.ds(c * CHU, CHU)], sems[b2])
                else:
                    for d in range(D):
                        pltpu.async_copy(
                            buf.at[pl.ds(b2 * BUFW + d * CHU, CHU)],
                            tf.at[pl.ds(d * nrows + c * CHU, CHU)], sems[b2])

        for s in range(nslot):
            if s >= 2:
                drain_writes(s)
            fire_reads(s)
            if s >= 1:
                process(s - 1)
        process(nslot - 1)
        for sl in (nslot - 2, nslot - 1):
            cl = sl * NW + wid
            b2 = sl & 1

            @pl.when(cl < nfull)
            def _dl():
                pltpu.make_async_copy(
                    tf.at[pl.ds(0, wwords)],
                    buf.at[pl.ds(b2 * BUFW, wwords)], sems[b2]).wait()

        @pl.when(wid == 0)
        def _tail0():
            rds = [pltpu.async_copy(tT.at[d, pl.ds(t0_off, t0_w)],
                                    tbuf.at[pl.ds(d * t0_w, t0_w)], semr)
                   for d in range(D)]
            for r in rds:
                r.wait()
            if rw is not None:
                dot_cols(rw, tbuf, 0, t0_w, tbuf, D * t0_w)
                pltpu.async_copy(tbuf.at[pl.ds(D * t0_w, t0_w)],
                                 tf.at[pl.ds(t0_off, t0_w)], semr).wait()
            else:
                wrs = [pltpu.async_copy(tbuf.at[pl.ds(d * t0_w, t0_w)],
                                        tf.at[pl.ds(d * nrows + t0_off, t0_w)],
                                        semr)
                       for d in range(D)]
                for w in wrs:
                    w.wait()

        @pl.when(wid == 1)
        def _tail1():
            rds = [pltpu.async_copy(tseg.at[pl.ds(d * t1_w, t1_w)],
                                    tbuf.at[pl.ds(d * t1_w, t1_w)], semr)
                   for d in range(D)]
            for r in rds:
                r.wait()
            if rw is not None:
                dot_cols(rw, tbuf, 0, t1_w, tbuf, D * t1_w)
                pltpu.async_copy(tbuf.at[pl.ds(D * t1_w, t1_w)],
                                 tf.at[pl.ds(t1_off, t1_w)], semr).wait()
            else:
                wrs = [pltpu.async_copy(tbuf.at[pl.ds(d * t1_w, t1_w)],
                                        tf.at[pl.ds(d * nrows + t1_off, t1_w)],
                                        semr)
                       for d in range(D)]
                for w in wrs:
                    w.wait()

    def do_table_rows(tT, tf, tseg):
        nslot = (NFULL_I + NW - 1) // NW
        lanes = lax.iota(_i32, L)
        ro = (0, BUFI)
        to = (2 * BUFI, 3 * BUFI)

        def transpose(roff, toff, n):
            def tg(g, carry):
                bidx = toff + g * (L * D) + lanes * D
                for d in range(D):
                    v = buf[pl.ds(roff + d * n + g * L, L)]
                    plsc.store_scatter(buf, [bidx + d], v)
                return carry
            lax.fori_loop(0, n // L, tg, 0)

        def fire_reads(s):
            c = s * NW + wid
            b2 = s & 1

            @pl.when(c < NFULL_I)
            def _():
                for d in range(D):
                    pltpu.async_copy(
                        tT.at[d, pl.ds(c * CHI, CHI)],
                        buf.at[pl.ds(ro[b2] + d * CHI, CHI)], semrs[b2])

        def drain_writes(s):
            cprev = (s - 2) * NW + wid
            b2 = s & 1

            @pl.when(cprev < NFULL_I)
            def _():
                pltpu.make_async_copy(
                    tf.at[pl.ds(0, BUFI)],
                    buf.at[pl.ds(to[b2], BUFI)], sems[b2]).wait()

        def process(s):
            c = s * NW + wid
            b2 = s & 1

            @pl.when(c < NFULL_I)
            def _():
                pltpu.make_async_copy(
                    tf.at[pl.ds(0, BUFI)],
                    buf.at[pl.ds(ro[b2], BUFI)], semrs[b2]).wait()
                transpose(ro[b2], to[b2], CHI)
                pltpu.async_copy(buf.at[pl.ds(to[b2], BUFI)],
                                 tf.at[pl.ds(c * BUFI, BUFI)], sems[b2])

        for s in range(nslot):
            if s >= 2:
                drain_writes(s)
            fire_reads(s)
            if s >= 1:
                process(s - 1)
        process(nslot - 1)
        for sl in (nslot - 2, nslot - 1):
            cl = sl * NW + wid
            b2 = sl & 1

            @pl.when(cl < NFULL_I)
            def _dl():
                pltpu.make_async_copy(
                    tf.at[pl.ds(0, BUFI)],
                    buf.at[pl.ds(to[b2], BUFI)], sems[b2]).wait()

        @pl.when(wid == 2)
        def _tail0():
            rds = [pltpu.async_copy(tT.at[d, pl.ds(T0I_OFF, 640)],
                                    tbuf.at[pl.ds(d * 640, 640)], semr)
                   for d in range(D)]
            for r in rds:
                r.wait()
            def tg(g, carry):
                bidx = D * 640 + g * (L * D) + lanes * D
                for d in range(D):
                    v = tbuf[pl.ds(d * 640 + g * L, L)]
                    plsc.store_scatter(tbuf, [bidx + d], v)
                return carry
            lax.fori_loop(0, 640 // L, tg, 0)
            pltpu.async_copy(tbuf.at[pl.ds(D * 640, 640 * D)],
                             tf.at[pl.ds(T0I_OFF * D, 640 * D)], semr).wait()

        @pl.when(wid == 3)
        def _tail1():
            pltpu.sync_copy(tseg, tbuf.at[pl.ds(0, 32 * D)])
            def tg(g, carry):
                bidx = 32 * D + g * (L * D) + lanes * D
                for d in range(D):
                    v = tbuf[pl.ds(d * 32 + g * L, L)]
                    plsc.store_scatter(tbuf, [bidx + d], v)
                return carry
            lax.fori_loop(0, 32 // L, tg, 0)
            pltpu.async_copy(tbuf.at[pl.ds(32 * D, 32 * D)],
                             tf.at[pl.ds(T1I_OFF * D, 32 * D)], semr).wait()

    do_table(cfuT_hbm, cfu_f, tseg_cfu, None, NFULL, NU, TAIL0, 512,
             TAIL1, 64)
    do_table(nnuT_hbm, nnd_f, tseg_nnu, wnnu, NFULL, NU, TAIL0, 512,
             TAIL1, 64)
    do_table(nniT_hbm, nnid_f, tseg_nni, wnni, NI // CHU, NI, (NI // CHU) * CHU,
             1664, 99968, 32)
    do_table_rows(cfiT_hbm, cfi_f, tseg_cfi)


_detile = functools.partial(
    pl.kernel,
    out_type=(jax.ShapeDtypeStruct((NU * D,), _f32),
              jax.ShapeDtypeStruct((NU,), _f32),
              jax.ShapeDtypeStruct((NI,), _f32),
              jax.ShapeDtypeStruct((NI * D,), _f32)),
    mesh=plsc.VectorSubcoreMesh(core_axis_name="c", subcore_axis_name="s",
                                num_cores=2, num_subcores=16),
    compiler_params=pltpu.CompilerParams(needs_layout_passes=False,
                                         use_tc_tiling_on_sc=True),
    scratch_types=[
        pltpu.VMEM((2 * BUFW + 2 * CHU,), _f32),
        pltpu.VMEM((D * 1664 + 1664,), _f32),
        pltpu.VMEM((8, 16), _f32),
        pltpu.VMEM((16, 32), _f32),
        pltpu.SemaphoreType.DMA,
        pltpu.SemaphoreType.DMA,
        pltpu.SemaphoreType.DMA,
        pltpu.SemaphoreType.DMA,
    ],
)(_detile_body)


def _main_body(xT_hbm, cfu_hbm, cfi_hbm, nnd_hbm, nnid_hbm, icW_hbm,
               ucW_hbm, par_hbm, out_hbm,
               xT_v, ug_v, us_v, ig_v, iq_v, isub_v, cfu_b, nnd_b, nnid_b,
               cfi_b, out_v, par_v, icW_v, ucW_v, sem):
    cid = lax.axis_index("c")
    sid = lax.axis_index("s")
    wid = cid * 16 + sid
    base = wid * RPW

    pltpu.sync_copy(par_hbm, par_v)
    pltpu.sync_copy(icW_hbm, icW_v)
    pltpu.sync_copy(ucW_hbm, ucW_v)
    pltpu.sync_copy(xT_hbm.at[:, pl.ds(base, RPW)], xT_v)

    lanes = lax.iota(_i32, L)

    def build(g, carry):
        sl = pl.ds(g * L, L)
        u = xT_v[0, sl].astype(_i32)
        i = xT_v[1, sl].astype(_i32)
        uq = jnp.right_shift(u, 4)
        for d in range(D):
            ug_v[d, sl] = uq + d * UROWS
        us_v[sl] = jnp.bitwise_and(u, 15)
        ig_v[sl] = i
        iq_v[sl] = jnp.right_shift(i, 4)
        isub_v[sl] = jnp.bitwise_and(i, 15)
        return carry
    lax.fori_loop(0, G, build, 0)

    ci1 = pltpu.async_copy(cfi_hbm.at[ig_v], cfi_b, sem)
    ci2 = pltpu.async_copy(nnid_hbm.at[iq_v], nnid_b, sem)

    pa_nn = par_v[0, :]
    pa_ic = par_v[1, :]
    pa_uc = par_v[2, :]
    pa_ab = par_v[3, :]
    wic0 = jnp.zeros((L,), _f32)
    wic1 = jnp.zeros((L,), _f32)
    wuc0 = jnp.zeros((L,), _f32)
    wuc1 = jnp.zeros((L,), _f32)
    for k in range(16):
        s_ic = pa_ic[k]
        wic0 = wic0 + s_ic * icW_v[k, pl.ds(0, L)]
        wic1 = wic1 + s_ic * icW_v[k, pl.ds(L, L)]
        s_uc = pa_uc[k]
        wuc0 = wuc0 + s_uc * ucW_v[k, pl.ds(0, L)]
        wuc1 = wuc1 + s_uc * ucW_v[k, pl.ds(L, L)]
    wfeat = (wic0, wic1, wuc0, wuc1)

    alpha = pa_ab[0]
    bias = (pa_ab[1]
            + jnp.sum(pa_nn * par_v[4, :])
            + jnp.sum(pa_ic * par_v[5, :])
            + jnp.sum(pa_uc * par_v[6, :]))

    def feats(g, carry):
        sl = pl.ds(g * L, L)
        acc = bias + jnp.zeros((L,), _f32)
        for c in range(4):
            for dd in range(16):
                d = c * 16 + dd
                acc = acc + wfeat[c][dd] * xT_v[2 + d, sl]
        out_v[sl] = acc
        return carry
    lax.fori_loop(0, G, feats, 0)

    ci1.wait()
    ci2.wait()

    def chunk(ch, carry):
        cb = ch * CB
        isl = pl.ds(cb, CB)
        cps = [pltpu.async_copy(nnd_hbm.at[ug_v.at[0, isl]], nnd_b, sem)]
        for d in range(D):
            cps.append(pltpu.async_copy(
                cfu_hbm.at[ug_v.at[d, isl]], cfu_b.at[d], sem))
        for c in cps:
            c.wait()
        for g2 in range(CB // L):
            rl = g2 * L + lanes
            sl = pl.ds(cb + g2 * L, L)
            usub = us_v[sl]
            acc = (out_v[sl] + plsc.load_gather(nnd_b, [rl, usub])
                   + plsc.load_gather(nnid_b, [cb + rl, isub_v[sl]]))
            cfacc = jnp.zeros((L,), _f32)
            for d in range(D):
                dcol = jnp.zeros((L,), _i32) + d
                cu = plsc.load_gather(cfu_b, [dcol, rl, usub])
                ci = plsc.load_gather(cfi_b, [cb + rl, dcol])
                cfacc = cfacc + cu * ci
            out_v[sl] = acc + alpha * cfacc
        return carry
    lax.fori_loop(0, NCB, chunk, 0)

    pltpu.sync_copy(out_v, out_hbm.at[pl.ds(base, RPW)])


_main = functools.partial(
    pl.kernel,
    out_type=jax.ShapeDtypeStruct((B,), _f32),
    mesh=plsc.VectorSubcoreMesh(core_axis_name="c", subcore_axis_name="s",
                                num_cores=2, num_subcores=16),
    compiler_params=pltpu.CompilerParams(needs_layout_passes=False,
                                         use_tc_tiling_on_sc=False),
    scratch_types=[
        pltpu.VMEM((XW, RPW), _f32),
        pltpu.VMEM((D, RPW), _i32),
        pltpu.VMEM((RPW,), _i32),
        pltpu.VMEM((RPW,), _i32),
        pltpu.VMEM((RPW,), _i32),
        pltpu.VMEM((RPW,), _i32),
        pltpu.VMEM((D, CB, L), _f32),
        pltpu.VMEM((CB, L), _f32),
        pltpu.VMEM((RPW, L), _f32),
        pltpu.VMEM((RPW, D), _f32),
        pltpu.VMEM((RPW,), _f32),
        pltpu.VMEM((8, 16), _f32),
        pltpu.VMEM((16, 32), _f32),
        pltpu.VMEM((16, 32), _f32),
        pltpu.SemaphoreType.DMA,
    ],
)(_main_body)


def kernel(x, cf_user_emb, cf_item_emb, nn_user_emb, nn_item_emb, nn_fc_W,
           nn_fc_b, ic_W, ic_b, uc_W, uc_b, fc_W, fc_b,
           item_context_features_in, user_context_features_in):
    row3 = jnp.concatenate([fc_W[0, 0:1], fc_b, jnp.zeros((14,), _f32)])
    params = jnp.stack([
        fc_W[0, 1:17], fc_W[0, 17:33], fc_W[0, 33:49], row3,
        nn_fc_b, ic_b, uc_b, jnp.zeros((16,), _f32),
    ])
    tseg_cfu = cf_user_emb[TAIL1:, :].T.reshape(-1)
    tseg_nnu = nn_user_emb[TAIL1:, :].T.reshape(-1)
    tseg_nni = nn_item_emb[99968:, :].T.reshape(-1)
    tseg_cfi = cf_item_emb[99968:, :].T.reshape(-1)
    cfu_f, nnd_f, nnid_f, cfi_f = _detile(
        cf_user_emb.T, nn_user_emb.T, nn_item_emb.T, cf_item_emb.T,
        tseg_cfu, tseg_nnu, tseg_nni, tseg_cfi, nn_fc_W, params)
    out = _main(x.T, cfu_f.reshape(NU, D), cfi_f.reshape(NI, D),
                nnd_f.reshape(NU // L, L), nnid_f.reshape(NI // L, L),
                ic_W, uc_W, params)
    return out[:, None]


# speedup vs baseline: 1.1690x; 1.0126x over previous
import functools

import jax
import jax.numpy as jnp
from jax import lax
from jax.experimental import pallas as pl
from jax.experimental.pallas import tpu as pltpu
from jax.experimental.pallas import tpu_sc as plsc

B = 16384
L = 16
NW = 32
RPW = B // NW
G = RPW // L
XW = 66
D = 16
NU = 1000000
NI = 100000

CHU = 2048
NFULL = NU // CHU
TAIL0 = NFULL * CHU
TAIL1 = TAIL0 + 512
BUFW = D * CHU

CHI = 1024
BUFI = D * CHI
NFULL_I = NI // CHI
T0I_OFF = NFULL_I * CHI
T1I_OFF = 99968

CB = 64
NCB = RPW // CB
UROWS = NU // L

_f32 = jnp.float32
_i32 = jnp.int32


def _detile_body(cfuT_hbm, nnuT_hbm, nniT_hbm, cfiT_hbm, tseg_cfu, tseg_nnu,
                 tseg_nni, tseg_cfi, nnW_hbm, par_hbm, cfu_f, nnd_f, nnid_f,
                 cfi_f, buf, tbuf, par_v, nnW_v, semr, semr2, semw0, semw1):
    cid = lax.axis_index("c")
    sid = lax.axis_index("s")
    wid = cid * 16 + sid
    sems = (semw0, semw1)
    semrs = (semr, semr2)

    pltpu.sync_copy(par_hbm, par_v)
    pltpu.sync_copy(nnW_hbm, nnW_v)
    pa_nn = par_v[0, :]
    wnnu = jnp.zeros((L,), _f32)
    wnni = jnp.zeros((L,), _f32)
    for k in range(16):
        wnnu = wnnu + pa_nn[k] * nnW_v[k, pl.ds(0, L)]
        wnni = wnni + pa_nn[k] * nnW_v[k, pl.ds(L, L)]

    def dot_cols(w, src, off, n, dst, dsl):
        def dot_g(g, carry):
            acc = jnp.zeros((L,), _f32)
            for d in range(D):
                acc = acc + w[d] * src[pl.ds(off + d * n + g * L, L)]
            dst[pl.ds(dsl + g * L, L)] = acc
            return carry
        lax.fori_loop(0, n // L, dot_g, 0)

    def do_table(tT, tf, tseg, rw, nfull, nrows, t0_off, t0_w, t1_off, t1_w):
        wwords = CHU if rw is not None else BUFW
        nslot = (nfull + NW - 1) // NW

        def fire_reads(s):
            c = s * NW + wid
            b2 = s & 1

            @pl.when(c < nfull)
            def _():
                for d in range(D):
                    pltpu.async_copy(
                        tT.at[d, pl.ds(c * CHU, CHU)],
                        buf.at[pl.ds(b2 * BUFW + d * CHU, CHU)], semrs[b2])

        def drain_writes(s):
            cprev = (s - 2) * NW + wid
            b2 = s & 1

            @pl.when(cprev < nfull)
            def _():
                pltpu.make_async_copy(
                    tf.at[pl.ds(0, wwords)],
                    buf.at[pl.ds(b2 * BUFW, wwords)], sems[b2]).wait()

        def process(s):
            c = s * NW + wid
            b2 = s & 1

            @pl.when(c < nfull)
            def _():
                pltpu.make_async_copy(
                    tf.at[pl.ds(0, BUFW)],
                    buf.at[pl.ds(b2 * BUFW, BUFW)], semrs[b2]).wait()
                if rw is not None:
                    dot_cols(rw, buf, b2 * BUFW, CHU, buf,
                             2 * BUFW + b2 * CHU)
                    pltpu.async_copy(
                        buf.at[pl.ds(2 * BUFW + b2 * CHU, CHU)],
                        tf.at[pl.ds(c * CHU, CHU)], sems[b2])
                else:
                    for d in range(D):
                        pltpu.async_copy(
                            buf.at[pl.ds(b2 * BUFW + d * CHU, CHU)],
                            tf.at[pl.ds(d * nrows + c * CHU, CHU)], sems[b2])

        for s in range(nslot):
            if s >= 2:
                drain_writes(s)
            fire_reads(s)
            if s >= 1:
                process(s - 1)
        process(nslot - 1)
        for sl in (nslot - 2, nslot - 1):
            cl = sl * NW + wid
            b2 = sl & 1

            @pl.when(cl < nfull)
            def _dl():
                pltpu.make_async_copy(
                    tf.at[pl.ds(0, wwords)],
                    buf.at[pl.ds(b2 * BUFW, wwords)], sems[b2]).wait()

        @pl.when(wid == 0)
        def _tail0():
            rds = [pltpu.async_copy(tT.at[d, pl.ds(t0_off, t0_w)],
                                    tbuf.at[pl.ds(d * t0_w, t0_w)], semr)
                   for d in range(D)]
            for r in rds:
                r.wait()
            if rw is not None:
                dot_cols(rw, tbuf, 0, t0_w, tbuf, D * t0_w)
                pltpu.async_copy(tbuf.at[pl.ds(D * t0_w, t0_w)],
                                 tf.at[pl.ds(t0_off, t0_w)], semr).wait()
            else:
                wrs = [pltpu.async_copy(tbuf.at[pl.ds(d * t0_w, t0_w)],
                                        tf.at[pl.ds(d * nrows + t0_off, t0_w)],
                                        semr)
                       for d in range(D)]
                for w in wrs:
                    w.wait()

        @pl.when(wid == 1)
        def _tail1():
            rds = [pltpu.async_copy(tseg.at[pl.ds(d * t1_w, t1_w)],
                                    tbuf.at[pl.ds(d * t1_w, t1_w)], semr)
                   for d in range(D)]
            for r in rds:
                r.wait()
            if rw is not None:
                dot_cols(rw, tbuf, 0, t1_w, tbuf, D * t1_w)
                pltpu.async_copy(tbuf.at[pl.ds(D * t1_w, t1_w)],
                                 tf.at[pl.ds(t1_off, t1_w)], semr).wait()
            else:
                wrs = [pltpu.async_copy(tbuf.at[pl.ds(d * t1_w, t1_w)],
                                        tf.at[pl.ds(d * nrows + t1_off, t1_w)],
                                        semr)
                       for d in range(D)]
                for w in wrs:
                    w.wait()

    def do_table_rows(tT, tf, tseg):
        nslot = (NFULL_I + NW - 1) // NW
        lanes = lax.iota(_i32, L)
        ro = (0, BUFI)
        to = (2 * BUFI, 3 * BUFI)

        def transpose(roff, toff, n):
            def tg(g, carry):
                bidx = toff + g * (L * D) + lanes * D
                for d in range(D):
                    v = buf[pl.ds(roff + d * n + g * L, L)]
                    plsc.store_scatter(buf, [bidx + d], v)
                return carry
            lax.fori_loop(0, n // L, tg, 0)

        def fire_reads(s):
            c = s * NW + wid
            b2 = s & 1

            @pl.when(c < NFULL_I)
            def _():
                for d in range(D):
                    pltpu.async_copy(
                        tT.at[d, pl.ds(c * CHI, CHI)],
                        buf.at[pl.ds(ro[b2] + d * CHI, CHI)], semrs[b2])

        def drain_writes(s):
            cprev = (s - 2) * NW + wid
            b2 = s & 1

            @pl.when(cprev < NFULL_I)
            def _():
                pltpu.make_async_copy(
                    tf.at[pl.ds(0, BUFI)],
                    buf.at[pl.ds(to[b2], BUFI)], sems[b2]).wait()

        def process(s):
            c = s * NW + wid
            b2 = s & 1

            @pl.when(c < NFULL_I)
            def _():
                pltpu.make_async_copy(
                    tf.at[pl.ds(0, BUFI)],
                    buf.at[pl.ds(ro[b2], BUFI)], semrs[b2]).wait()
                transpose(ro[b2], to[b2], CHI)
                pltpu.async_copy(buf.at[pl.ds(to[b2], BUFI)],
                                 tf.at[pl.ds(c * BUFI, BUFI)], sems[b2])

        for s in range(nslot):
            if s >= 2:
                drain_writes(s)
            fire_reads(s)
            if s >= 1:
                process(s - 1)
        process(nslot - 1)
        for sl in (nslot - 2, nslot - 1):
            cl = sl * NW + wid
            b2 = sl & 1

            @pl.when(cl < NFULL_I)
            def _dl():
                pltpu.make_async_copy(
                    tf.at[pl.ds(0, BUFI)],
                    buf.at[pl.ds(to[b2], BUFI)], sems[b2]).wait()

        @pl.when(wid == 2)
        def _tail0():
            rds = [pltpu.async_copy(tT.at[d, pl.ds(T0I_OFF, 640)],
                                    tbuf.at[pl.ds(d * 640, 640)], semr)
                   for d in range(D)]
            for r in rds:
                r.wait()
            def tg(g, carry):
                bidx = D * 640 + g * (L * D) + lanes * D
                for d in range(D):
                    v = tbuf[pl.ds(d * 640 + g * L, L)]
                    plsc.store_scatter(tbuf, [bidx + d], v)
                return carry
            lax.fori_loop(0, 640 // L, tg, 0)
            pltpu.async_copy(tbuf.at[pl.ds(D * 640, 640 * D)],
                             tf.at[pl.ds(T0I_OFF * D, 640 * D)], semr).wait()

        @pl.when(wid == 3)
        def _tail1():
            pltpu.sync_copy(tseg, tbuf.at[pl.ds(0, 32 * D)])
            def tg(g, carry):
                bidx = 32 * D + g * (L * D) + lanes * D
                for d in range(D):
                    v = tbuf[pl.ds(d * 32 + g * L, L)]
                    plsc.store_scatter(tbuf, [bidx + d], v)
                return carry
            lax.fori_loop(0, 32 // L, tg, 0)
            pltpu.async_copy(tbuf.at[pl.ds(32 * D, 32 * D)],
                             tf.at[pl.ds(T1I_OFF * D, 32 * D)], semr).wait()

    do_table(cfuT_hbm, cfu_f, tseg_cfu, None, NFULL, NU, TAIL0, 512,
             TAIL1, 64)
    do_table(nnuT_hbm, nnd_f, tseg_nnu, wnnu, NFULL, NU, TAIL0, 512,
             TAIL1, 64)
    do_table(nniT_hbm, nnid_f, tseg_nni, wnni, NI // CHU, NI, (NI // CHU) * CHU,
             1664, 99968, 32)
    do_table_rows(cfiT_hbm, cfi_f, tseg_cfi)


_detile = functools.partial(
    pl.kernel,
    out_type=(jax.ShapeDtypeStruct((NU * D,), _f32),
              jax.ShapeDtypeStruct((NU,), _f32),
              jax.ShapeDtypeStruct((NI,), _f32),
              jax.ShapeDtypeStruct((NI * D,), _f32)),
    mesh=plsc.VectorSubcoreMesh(core_axis_name="c", subcore_axis_name="s",
                                num_cores=2, num_subcores=16),
    compiler_params=pltpu.CompilerParams(needs_layout_passes=False,
                                         use_tc_tiling_on_sc=True),
    scratch_types=[
        pltpu.VMEM((2 * BUFW + 2 * CHU,), _f32),
        pltpu.VMEM((D * 1664 + 1664,), _f32),
        pltpu.VMEM((8, 16), _f32),
        pltpu.VMEM((16, 32), _f32),
        pltpu.SemaphoreType.DMA,
        pltpu.SemaphoreType.DMA,
        pltpu.SemaphoreType.DMA,
        pltpu.SemaphoreType.DMA,
    ],
)(_detile_body)


def _main_body(xT_hbm, cfu_hbm, cfi_hbm, nnd_hbm, nnid_hbm, icW_hbm,
               ucW_hbm, par_hbm, out_hbm,
               xT_v, ug_v, us_v, ig_v, iq_v, isub_v, cfu_b, nnd_b, nnid_b,
               cfi_b, out_v, par_v, icW_v, ucW_v, sem, sem2):
    cid = lax.axis_index("c")
    sid = lax.axis_index("s")
    wid = cid * 16 + sid
    base = wid * RPW

    pltpu.sync_copy(par_hbm, par_v)
    pltpu.sync_copy(icW_hbm, icW_v)
    pltpu.sync_copy(ucW_hbm, ucW_v)
    pltpu.sync_copy(xT_hbm.at[:, pl.ds(base, RPW)], xT_v)

    lanes = lax.iota(_i32, L)

    def build(g, carry):
        sl = pl.ds(g * L, L)
        u = xT_v[0, sl].astype(_i32)
        i = xT_v[1, sl].astype(_i32)
        uq = jnp.right_shift(u, 4)
        for d in range(D):
            ug_v[d, sl] = uq + d * UROWS
        us_v[sl] = jnp.bitwise_and(u, 15)
        ig_v[sl] = i
        iq_v[sl] = jnp.right_shift(i, 4)
        isub_v[sl] = jnp.bitwise_and(i, 15)
        return carry
    lax.fori_loop(0, G, build, 0)

    ci1 = pltpu.async_copy(cfi_hbm.at[ig_v], cfi_b, sem)
    ci2 = pltpu.async_copy(nnid_hbm.at[iq_v], nnid_b, sem)

    pa_nn = par_v[0, :]
    pa_ic = par_v[1, :]
    pa_uc = par_v[2, :]
    pa_ab = par_v[3, :]
    wic0 = jnp.zeros((L,), _f32)
    wic1 = jnp.zeros((L,), _f32)
    wuc0 = jnp.zeros((L,), _f32)
    wuc1 = jnp.zeros((L,), _f32)
    for k in range(16):
        s_ic = pa_ic[k]
        wic0 = wic0 + s_ic * icW_v[k, pl.ds(0, L)]
        wic1 = wic1 + s_ic * icW_v[k, pl.ds(L, L)]
        s_uc = pa_uc[k]
        wuc0 = wuc0 + s_uc * ucW_v[k, pl.ds(0, L)]
        wuc1 = wuc1 + s_uc * ucW_v[k, pl.ds(L, L)]
    wfeat = (wic0, wic1, wuc0, wuc1)

    alpha = pa_ab[0]
    bias = (pa_ab[1]
            + jnp.sum(pa_nn * par_v[4, :])
            + jnp.sum(pa_ic * par_v[5, :])
            + jnp.sum(pa_uc * par_v[6, :]))

    def feats(g, carry):
        sl = pl.ds(g * L, L)
        acc = bias + jnp.zeros((L,), _f32)
        for c in range(4):
            for dd in range(16):
                d = c * 16 + dd
                acc = acc + wfeat[c][dd] * xT_v[2 + d, sl]
        out_v[sl] = acc
        return carry
    lax.fori_loop(0, G, feats, 0)

    ci1.wait()
    ci2.wait()

    sems = (sem, sem2)

    def fire(ch):
        b2 = ch & 1
        isl = pl.ds(ch * CB, CB)
        cps = [pltpu.async_copy(nnd_hbm.at[ug_v.at[0, isl]], nnd_b.at[b2],
                                sems[b2])]
        for d in range(D):
            cps.append(pltpu.async_copy(
                cfu_hbm.at[ug_v.at[d, isl]], cfu_b.at[b2, d], sems[b2]))
        return cps

    pend = fire(0)
    for ch in range(NCB):
        b2 = ch & 1
        nxt = fire(ch + 1) if ch + 1 < NCB else []
        for c in pend:
            c.wait()
        cb = ch * CB
        for g2 in range(CB // L):
            rl = g2 * L + lanes
            sl = pl.ds(cb + g2 * L, L)
            usub = us_v[sl]
            acc = (out_v[sl] + plsc.load_gather(nnd_b.at[b2], [rl, usub])
                   + plsc.load_gather(nnid_b, [cb + rl, isub_v[sl]]))
            cfacc = jnp.zeros((L,), _f32)
            for d in range(D):
                dcol = jnp.zeros((L,), _i32) + d
                cu = plsc.load_gather(cfu_b.at[b2], [dcol, rl, usub])
                ci = plsc.load_gather(cfi_b, [cb + rl, dcol])
                cfacc = cfacc + cu * ci
            out_v[sl] = acc + alpha * cfacc
        pend = nxt

    pltpu.sync_copy(out_v, out_hbm.at[pl.ds(base, RPW)])


_main = functools.partial(
    pl.kernel,
    out_type=jax.ShapeDtypeStruct((B,), _f32),
    mesh=plsc.VectorSubcoreMesh(core_axis_name="c", subcore_axis_name="s",
                                num_cores=2, num_subcores=16),
    compiler_params=pltpu.CompilerParams(needs_layout_passes=False,
                                         use_tc_tiling_on_sc=False),
    scratch_types=[
        pltpu.VMEM((XW, RPW), _f32),
        pltpu.VMEM((D, RPW), _i32),
        pltpu.VMEM((RPW,), _i32),
        pltpu.VMEM((RPW,), _i32),
        pltpu.VMEM((RPW,), _i32),
        pltpu.VMEM((RPW,), _i32),
        pltpu.VMEM((2, D, CB, L), _f32),
        pltpu.VMEM((2, CB, L), _f32),
        pltpu.VMEM((RPW, L), _f32),
        pltpu.VMEM((RPW, D), _f32),
        pltpu.VMEM((RPW,), _f32),
        pltpu.VMEM((8, 16), _f32),
        pltpu.VMEM((16, 32), _f32),
        pltpu.VMEM((16, 32), _f32),
        pltpu.SemaphoreType.DMA,
        pltpu.SemaphoreType.DMA,
    ],
)(_main_body)


def kernel(x, cf_user_emb, cf_item_emb, nn_user_emb, nn_item_emb, nn_fc_W,
           nn_fc_b, ic_W, ic_b, uc_W, uc_b, fc_W, fc_b,
           item_context_features_in, user_context_features_in):
    row3 = jnp.concatenate([fc_W[0, 0:1], fc_b, jnp.zeros((14,), _f32)])
    params = jnp.stack([
        fc_W[0, 1:17], fc_W[0, 17:33], fc_W[0, 33:49], row3,
        nn_fc_b, ic_b, uc_b, jnp.zeros((16,), _f32),
    ])
    tseg_cfu = cf_user_emb[TAIL1:, :].T.reshape(-1)
    tseg_nnu = nn_user_emb[TAIL1:, :].T.reshape(-1)
    tseg_nni = nn_item_emb[99968:, :].T.reshape(-1)
    tseg_cfi = cf_item_emb[99968:, :].T.reshape(-1)
    cfu_f, nnd_f, nnid_f, cfi_f = _detile(
        cf_user_emb.T, nn_user_emb.T, nn_item_emb.T, cf_item_emb.T,
        tseg_cfu, tseg_nnu, tseg_nni, tseg_cfi, nn_fc_W, params)
    out = _main(x.T, cfu_f.reshape(NU, D), cfi_f.reshape(NI, D),
                nnd_f.reshape(NU // L, L), nnid_f.reshape(NI // L, L),
                ic_W, uc_W, params)
    return out[:, None]
